# baseline (device time: 20056 ns/iter reference)
import os

import jax
import jax.numpy as jnp
from jax import lax
from jax.experimental import pallas as pl
from jax.experimental.pallas import tpu as pltpu

N_DEV = 4
B_PER = 2
SQ = 128
SKV = 128
HQ = 16
H_PER = HQ // N_DEV
DH = 64
DM = 512
DG = H_PER * DH

DO_COMM = os.environ.get("ABL_NO_COMM", "") != "1"


def kernel(x, Wq, K_ext, V_ext, Wo):
    my_out = lax.axis_index("i")
    K_loc = lax.dynamic_slice_in_dim(
        K_ext, my_out * B_PER, B_PER, axis=0).astype(jnp.bfloat16).reshape(
            B_PER, SKV, N_DEV, DG).transpose(2, 0, 1, 3)
    V_loc = lax.dynamic_slice_in_dim(
        V_ext, my_out * B_PER, B_PER, axis=0).astype(jnp.bfloat16).reshape(
            B_PER, SKV, N_DEV, DG).transpose(2, 0, 1, 3)

    def body(x_ref, wq_ref, k_ref, v_ref, wo_ref, out_ref,
             wq_comm, wo_comm, q_scr, ctx_scr, send_sems, recv_sems):
        my = lax.axis_index("i")

        wq_comm[pl.ds(my, 1)] = wq_ref[...].astype(jnp.bfloat16)[None]
        wo_comm[pl.ds(my, 1)] = wo_ref[...].astype(jnp.bfloat16)[None]

        barrier = pltpu.get_barrier_semaphore() if DO_COMM else None
        for o in range(1, N_DEV) if DO_COMM else []:
            peer = lax.rem(my + o, N_DEV)
            pl.semaphore_signal(barrier, inc=1, device_id=(peer,),
                                device_id_type=pl.DeviceIdType.MESH)
        if DO_COMM:
            pl.semaphore_wait(barrier, N_DEV - 1)

        sends = []
        for j, comm in enumerate((wq_comm, wo_comm)) if DO_COMM else []:
            for o in range(1, N_DEV):
                tgt = lax.rem(my + o, N_DEV)
                idx = (o - 1) + 3 * j
                rdma = pltpu.make_async_remote_copy(
                    src_ref=comm.at[my],
                    dst_ref=comm.at[my],
                    send_sem=send_sems.at[idx],
                    recv_sem=recv_sems.at[idx],
                    device_id=(tgt,),
                    device_id_type=pl.DeviceIdType.MESH,
                )
                rdma.start()
                sends.append(rdma)

        def wait_recvs(j, comm):
            if not DO_COMM:
                return
            for o in range(1, N_DEV):
                src = lax.rem(my - o + N_DEV, N_DEV)
                idx = (o - 1) + 3 * j
                rdma = pltpu.make_async_remote_copy(
                    src_ref=comm.at[src],
                    dst_ref=comm.at[src],
                    send_sem=send_sems.at[idx],
                    recv_sem=recv_sems.at[idx],
                    device_id=(src,),
                    device_id_type=pl.DeviceIdType.MESH,
                )
                rdma.wait_recv()

        xb = x_ref[...].reshape(B_PER * SQ, DM).astype(jnp.bfloat16)
        qb_blk = lax.broadcasted_iota(jnp.int32, (SQ, SKV), 0) // 64
        kb_blk = lax.broadcasted_iota(jnp.int32, (SQ, SKV), 1) // 64
        keep = (qb_blk == kb_blk) | ((kb_blk % 4) == (qb_blk % 4))
        mask_bias = jnp.where(keep, jnp.float32(0.0), jnp.float32(-1e9))

        def group_ctx(wq_g, kb, vb, slot):
            q = jnp.dot(xb, wq_g, preferred_element_type=jnp.float32)
            q_scr[...] = (q * 0.125).astype(jnp.bfloat16)
            for b in range(B_PER):
                for hl in range(H_PER):
                    qbh = q_scr[b * SQ:(b + 1) * SQ, hl * DH:(hl + 1) * DH]
                    kbh = kb[b, :, hl * DH:(hl + 1) * DH]
                    s = lax.dot_general(
                        qbh, kbh, (((1,), (1,)), ((), ())),
                        preferred_element_type=jnp.float32)
                    w = jnp.exp(s + mask_bias)
                    inv = 1.0 / jnp.sum(w, axis=-1, keepdims=True)
                    vbh = vb[b, :, hl * DH:(hl + 1) * DH]
                    c = jnp.dot(w.astype(jnp.bfloat16), vbh,
                                preferred_element_type=jnp.float32) * inv
                    ctx_scr[pl.ds(slot, 1), b * SQ:(b + 1) * SQ,
                            hl * DH:(hl + 1) * DH] = c.astype(jnp.bfloat16)[None]

        wait_recvs(0, wq_comm)
        for g in range(N_DEV):
            group_ctx(wq_comm[g], k_ref[g], v_ref[g], g)

        wait_recvs(1, wo_comm)
        acc = jnp.dot(ctx_scr[0], wo_comm[0], preferred_element_type=jnp.float32)
        for g in range(1, N_DEV):
            acc = acc + jnp.dot(ctx_scr[g], wo_comm[g],
                                preferred_element_type=jnp.float32)
        out_ref[...] = acc.reshape(B_PER, SQ, DM)

        for r in sends:
            r.wait_send()

    out_shape = jax.ShapeDtypeStruct((B_PER, SQ, DM), jnp.float32)
    vmem = pl.BlockSpec(memory_space=pltpu.MemorySpace.VMEM)
    return pl.pallas_call(
        body,
        out_shape=out_shape,
        in_specs=[vmem] * 5,
        out_specs=vmem,
        scratch_shapes=[
            pltpu.VMEM((N_DEV, DM, DG), jnp.bfloat16),
            pltpu.VMEM((N_DEV, DG, DM), jnp.bfloat16),
            pltpu.VMEM((B_PER * SQ, DG), jnp.bfloat16),
            pltpu.VMEM((N_DEV, B_PER * SQ, DG), jnp.bfloat16),
            pltpu.SemaphoreType.DMA((6,)),
            pltpu.SemaphoreType.DMA((6,)),
        ],
        compiler_params=pltpu.CompilerParams(
            collective_id=0 if DO_COMM else None),
    )(x, Wq, K_loc, V_loc, Wo)


# device time: 19973 ns/iter; 1.0042x vs baseline; 1.0042x over previous
import os

import jax
import jax.numpy as jnp
from jax import lax
from jax.experimental import pallas as pl
from jax.experimental.pallas import tpu as pltpu

N_DEV = 4
B_PER = 2
SQ = 128
SKV = 128
HQ = 16
H_PER = HQ // N_DEV
DH = 64
DM = 512
DG = H_PER * DH

DO_COMM = os.environ.get("ABL_NO_COMM", "") != "1"


def kernel(x, Wq, K_ext, V_ext, Wo):
    my_out = lax.axis_index("i")
    K_loc = lax.dynamic_slice_in_dim(
        K_ext, my_out * B_PER, B_PER, axis=0).astype(jnp.bfloat16).reshape(
            B_PER, SKV, N_DEV, DG).transpose(2, 0, 1, 3)
    V_loc = lax.dynamic_slice_in_dim(
        V_ext, my_out * B_PER, B_PER, axis=0).astype(jnp.bfloat16).reshape(
            B_PER, SKV, N_DEV, DG).transpose(2, 0, 1, 3)

    def body(x_ref, wq_ref, k_ref, v_ref, wo_ref, out_ref,
             wq_comm, wo_comm, q_scr, ctx_scr, send_sems, recv_sems):
        my = lax.axis_index("i")

        wq_comm[pl.ds(my, 1)] = wq_ref[...].astype(jnp.bfloat16)[None]
        wo_comm[pl.ds(my, 1)] = wo_ref[...].astype(jnp.bfloat16)[None]

        barrier = pltpu.get_barrier_semaphore() if DO_COMM else None
        for o in range(1, N_DEV) if DO_COMM else []:
            peer = lax.rem(my + o, N_DEV)
            pl.semaphore_signal(barrier, inc=1, device_id=(peer,),
                                device_id_type=pl.DeviceIdType.MESH)
        if DO_COMM:
            pl.semaphore_wait(barrier, N_DEV - 1)

        sends = []
        for j, comm in enumerate((wq_comm, wo_comm)) if DO_COMM else []:
            for o in range(1, N_DEV):
                tgt = lax.rem(my + o, N_DEV)
                idx = (o - 1) + 3 * j
                rdma = pltpu.make_async_remote_copy(
                    src_ref=comm.at[my],
                    dst_ref=comm.at[my],
                    send_sem=send_sems.at[idx],
                    recv_sem=recv_sems.at[idx],
                    device_id=(tgt,),
                    device_id_type=pl.DeviceIdType.MESH,
                )
                rdma.start()
                sends.append(rdma)

        def wait_recvs(j, comm):
            if not DO_COMM:
                return
            for o in range(1, N_DEV):
                src = lax.rem(my - o + N_DEV, N_DEV)
                idx = (o - 1) + 3 * j
                rdma = pltpu.make_async_remote_copy(
                    src_ref=comm.at[src],
                    dst_ref=comm.at[src],
                    send_sem=send_sems.at[idx],
                    recv_sem=recv_sems.at[idx],
                    device_id=(src,),
                    device_id_type=pl.DeviceIdType.MESH,
                )
                rdma.wait_recv()

        xb = x_ref[...].reshape(B_PER * SQ, DM).astype(jnp.bfloat16)
        qb_blk = lax.broadcasted_iota(jnp.int32, (SQ, SKV), 0) // 64
        kb_blk = lax.broadcasted_iota(jnp.int32, (SQ, SKV), 1) // 64
        keep = (qb_blk == kb_blk) | ((kb_blk % 4) == (qb_blk % 4))
        mask_bias = jnp.where(keep, jnp.float32(0.0), jnp.float32(-1e9))

        def group_ctx(wq_g, kb, vb, slot):
            q = jnp.dot(xb, wq_g, preferred_element_type=jnp.float32)
            q_scr[...] = (q * 0.125).astype(jnp.bfloat16)
            for b in range(B_PER):
                for hl in range(H_PER):
                    qbh = q_scr[b * SQ:(b + 1) * SQ, hl * DH:(hl + 1) * DH]
                    kbh = kb[b, :, hl * DH:(hl + 1) * DH]
                    s = lax.dot_general(
                        qbh, kbh, (((1,), (1,)), ((), ())),
                        preferred_element_type=jnp.float32)
                    w = jnp.exp(s + mask_bias)
                    inv = 1.0 / jnp.sum(w, axis=-1, keepdims=True)
                    vbh = vb[b, :, hl * DH:(hl + 1) * DH]
                    c = jnp.dot(w.astype(jnp.bfloat16), vbh,
                                preferred_element_type=jnp.float32) * inv
                    ctx_scr[pl.ds(slot, 1), b * SQ:(b + 1) * SQ,
                            hl * DH:(hl + 1) * DH] = c.astype(jnp.bfloat16)[None]

        for g in range(N_DEV):
            if DO_COMM:
                @pl.when(my != g)
                def _(g=g):
                    o_idx = lax.rem(my - g + N_DEV, N_DEV) - 1
                    rdma = pltpu.make_async_remote_copy(
                        src_ref=wq_comm.at[g],
                        dst_ref=wq_comm.at[g],
                        send_sem=send_sems.at[o_idx],
                        recv_sem=recv_sems.at[o_idx],
                        device_id=(g,),
                        device_id_type=pl.DeviceIdType.MESH,
                    )
                    rdma.wait_recv()
            group_ctx(wq_comm[g], k_ref[g], v_ref[g], g)

        wait_recvs(1, wo_comm)
        acc = jnp.dot(ctx_scr[0], wo_comm[0], preferred_element_type=jnp.float32)
        for g in range(1, N_DEV):
            acc = acc + jnp.dot(ctx_scr[g], wo_comm[g],
                                preferred_element_type=jnp.float32)
        out_ref[...] = acc.reshape(B_PER, SQ, DM)

        for r in sends:
            r.wait_send()

    out_shape = jax.ShapeDtypeStruct((B_PER, SQ, DM), jnp.float32)
    vmem = pl.BlockSpec(memory_space=pltpu.MemorySpace.VMEM)
    return pl.pallas_call(
        body,
        out_shape=out_shape,
        in_specs=[vmem] * 5,
        out_specs=vmem,
        scratch_shapes=[
            pltpu.VMEM((N_DEV, DM, DG), jnp.bfloat16),
            pltpu.VMEM((N_DEV, DG, DM), jnp.bfloat16),
            pltpu.VMEM((B_PER * SQ, DG), jnp.bfloat16),
            pltpu.VMEM((N_DEV, B_PER * SQ, DG), jnp.bfloat16),
            pltpu.SemaphoreType.DMA((6,)),
            pltpu.SemaphoreType.DMA((6,)),
        ],
        compiler_params=pltpu.CompilerParams(
            collective_id=0 if DO_COMM else None),
    )(x, Wq, K_loc, V_loc, Wo)


# device time: 19955 ns/iter; 1.0051x vs baseline; 1.0009x over previous
import os

import jax
import jax.numpy as jnp
from jax import lax
from jax.experimental import pallas as pl
from jax.experimental.pallas import tpu as pltpu

N_DEV = 4
B_PER = 2
SQ = 128
SKV = 128
HQ = 16
H_PER = HQ // N_DEV
DH = 64
DM = 512
DG = H_PER * DH

DO_COMM = os.environ.get("ABL_NO_COMM", "") != "1"


def kernel(x, Wq, K_ext, V_ext, Wo):
    my_out = lax.axis_index("i")
    K_loc = lax.dynamic_slice_in_dim(
        K_ext, my_out * B_PER, B_PER, axis=0).astype(jnp.bfloat16).reshape(
            B_PER, SKV, N_DEV, DG).transpose(2, 0, 1, 3)
    V_loc = lax.dynamic_slice_in_dim(
        V_ext, my_out * B_PER, B_PER, axis=0).astype(jnp.bfloat16).reshape(
            B_PER, SKV, N_DEV, DG).transpose(2, 0, 1, 3)

    def body(x_ref, wq_ref, k_ref, v_ref, wo_ref, out_ref,
             wq_comm, wo_comm, q_scr, ctx_scr, send_sems, recv_sems):
        my = lax.axis_index("i")

        wq_comm[pl.ds(my, 1)] = wq_ref[...].astype(jnp.bfloat16)[None]
        wo_comm[pl.ds(my, 1)] = wo_ref[...].astype(jnp.bfloat16)[None]

        barrier = pltpu.get_barrier_semaphore() if DO_COMM else None
        for o in range(1, N_DEV) if DO_COMM else []:
            peer = lax.rem(my + o, N_DEV)
            pl.semaphore_signal(barrier, inc=1, device_id=(peer,),
                                device_id_type=pl.DeviceIdType.MESH)
        if DO_COMM:
            pl.semaphore_wait(barrier, N_DEV - 1)

        sends = []
        for j, comm in enumerate((wq_comm, wo_comm)) if DO_COMM else []:
            for o in range(1, N_DEV):
                tgt = lax.rem(my + o, N_DEV)
                idx = (o - 1) + 3 * j
                rdma = pltpu.make_async_remote_copy(
                    src_ref=comm.at[my],
                    dst_ref=comm.at[my],
                    send_sem=send_sems.at[idx],
                    recv_sem=recv_sems.at[idx],
                    device_id=(tgt,),
                    device_id_type=pl.DeviceIdType.MESH,
                )
                rdma.start()
                sends.append(rdma)

        def wait_recvs(j, comm):
            if not DO_COMM:
                return
            for o in range(1, N_DEV):
                src = lax.rem(my - o + N_DEV, N_DEV)
                idx = (o - 1) + 3 * j
                rdma = pltpu.make_async_remote_copy(
                    src_ref=comm.at[src],
                    dst_ref=comm.at[src],
                    send_sem=send_sems.at[idx],
                    recv_sem=recv_sems.at[idx],
                    device_id=(src,),
                    device_id_type=pl.DeviceIdType.MESH,
                )
                rdma.wait_recv()

        xb = (x_ref[...].reshape(B_PER * SQ, DM) * 0.125).astype(jnp.bfloat16)
        qb_blk = lax.broadcasted_iota(jnp.int32, (SQ, SKV), 0) // 64
        kb_blk = lax.broadcasted_iota(jnp.int32, (SQ, SKV), 1) // 64
        keep = (qb_blk == kb_blk) | ((kb_blk % 4) == (qb_blk % 4))
        mask_bias = jnp.where(keep, jnp.float32(0.0), jnp.float32(-1e9))

        def group_ctx(wq_g, kb, vb, slot):
            q = jnp.dot(xb, wq_g, preferred_element_type=jnp.float32)
            q_scr[...] = q.astype(jnp.bfloat16)
            for b in range(B_PER):
                for hl in range(H_PER):
                    qbh = q_scr[b * SQ:(b + 1) * SQ, hl * DH:(hl + 1) * DH]
                    kbh = kb[b, :, hl * DH:(hl + 1) * DH]
                    s = lax.dot_general(
                        qbh, kbh, (((1,), (1,)), ((), ())),
                        preferred_element_type=jnp.float32)
                    w = jnp.exp(s + mask_bias)
                    inv = 1.0 / jnp.sum(w, axis=-1, keepdims=True)
                    vbh = vb[b, :, hl * DH:(hl + 1) * DH]
                    c = jnp.dot(w.astype(jnp.bfloat16), vbh,
                                preferred_element_type=jnp.float32) * inv
                    ctx_scr[pl.ds(slot, 1), b * SQ:(b + 1) * SQ,
                            hl * DH:(hl + 1) * DH] = c.astype(jnp.bfloat16)[None]

        for g in range(N_DEV):
            if DO_COMM:
                @pl.when(my != g)
                def _(g=g):
                    o_idx = lax.rem(my - g + N_DEV, N_DEV) - 1
                    rdma = pltpu.make_async_remote_copy(
                        src_ref=wq_comm.at[g],
                        dst_ref=wq_comm.at[g],
                        send_sem=send_sems.at[o_idx],
                        recv_sem=recv_sems.at[o_idx],
                        device_id=(g,),
                        device_id_type=pl.DeviceIdType.MESH,
                    )
                    rdma.wait_recv()
            group_ctx(wq_comm[g], k_ref[g], v_ref[g], g)

        wait_recvs(1, wo_comm)
        acc = jnp.dot(ctx_scr[0], wo_comm[0], preferred_element_type=jnp.float32)
        for g in range(1, N_DEV):
            acc = acc + jnp.dot(ctx_scr[g], wo_comm[g],
                                preferred_element_type=jnp.float32)
        out_ref[...] = acc.reshape(B_PER, SQ, DM)

        for r in sends:
            r.wait_send()

    out_shape = jax.ShapeDtypeStruct((B_PER, SQ, DM), jnp.float32)
    vmem = pl.BlockSpec(memory_space=pltpu.MemorySpace.VMEM)
    return pl.pallas_call(
        body,
        out_shape=out_shape,
        in_specs=[vmem] * 5,
        out_specs=vmem,
        scratch_shapes=[
            pltpu.VMEM((N_DEV, DM, DG), jnp.bfloat16),
            pltpu.VMEM((N_DEV, DG, DM), jnp.bfloat16),
            pltpu.VMEM((B_PER * SQ, DG), jnp.bfloat16),
            pltpu.VMEM((N_DEV, B_PER * SQ, DG), jnp.bfloat16),
            pltpu.SemaphoreType.DMA((6,)),
            pltpu.SemaphoreType.DMA((6,)),
        ],
        compiler_params=pltpu.CompilerParams(
            collective_id=0 if DO_COMM else None),
    )(x, Wq, K_loc, V_loc, Wo)
